# Initial kernel scaffold; baseline (speedup 1.0000x reference)
import functools

import jax
import jax.numpy as jnp
import numpy as np
from jax import lax
from jax.experimental import pallas as pl
from jax.experimental.pallas import tpu as pltpu
from jax.experimental.pallas import tpu_sc as plsc

_BATCH = 4096
_NFIELD = 30
_EMBED = 64
_TABLE = 30000
_KEPT_COLS = np.array(
    [0, 1, 2, 4, 5, 6, 7, 10, 11, 12, 13, 14, 17, 18, 21, 22, 23]
    + list(range(26, 39)),
    dtype=np.int32,
)
_OFFSETS = (np.arange(_NFIELD, dtype=np.int32) * 1000)

_NC, _NS = 2, 16
_NW = _NC * _NS
_NIDX = _BATCH * _NFIELD
_IDX_W = _NIDX // _NW
_CHUNK = 128
_NCHUNK = _IDX_W // _CHUNK
_SAMP_W = _BATCH // _NW

_BN_C = float(1.0 / np.sqrt(1.0 + 1e-5))


def _sc_gather(idx2d_hbm, emb_hbm, linw_hbm, rows_out, lin_out,
               idx_v, rows_v, lin_v, linw_v, sem):
    wid = lax.axis_index("s") * _NC + lax.axis_index("c")
    pltpu.sync_copy(idx2d_hbm.at[pl.ds(wid * _NCHUNK, _NCHUNK)], idx_v)
    pltpu.sync_copy(linw_hbm, linw_v)

    def chunk_body(c, carry):
        pltpu.async_copy(emb_hbm.at[idx_v.at[c]], rows_v, sem).wait()
        pltpu.sync_copy(
            rows_v, rows_out.at[pl.ds(wid * _IDX_W + c * _CHUNK, _CHUNK)])
        return carry

    lax.fori_loop(0, _NCHUNK, chunk_body, 0)

    lanes = lax.iota(jnp.int32, 16)

    def samp_body(g, carry):
        def field_body(f, acc):
            p = g * (16 * _NFIELD) + f + lanes * _NFIELD
            idxs = plsc.load_gather(idx_v, [p // _CHUNK, p % _CHUNK])
            return acc + plsc.load_gather(linw_v, [idxs])

        acc = lax.fori_loop(0, _NFIELD, field_body,
                            jnp.zeros((16,), jnp.float32))
        lin_v[pl.ds(g * 16, 16)] = acc
        return carry

    lax.fori_loop(0, _SAMP_W // 16, samp_body, 0)
    pltpu.sync_copy(lin_v, lin_out.at[pl.ds(wid * _SAMP_W, _SAMP_W)])


_sc_gather_call = functools.partial(
    pl.kernel,
    out_type=[
        jax.ShapeDtypeStruct((_NIDX, _EMBED), jnp.float32),
        jax.ShapeDtypeStruct((_BATCH,), jnp.float32),
    ],
    mesh=plsc.VectorSubcoreMesh(
        core_axis_name="c", subcore_axis_name="s",
        num_cores=_NC, num_subcores=_NS),
    scratch_types=[
        pltpu.VMEM((_NCHUNK, _CHUNK), jnp.int32),
        pltpu.VMEM((_CHUNK, _EMBED), jnp.float32),
        pltpu.VMEM((_SAMP_W,), jnp.float32),
        pltpu.VMEM((_TABLE,), jnp.float32),
        pltpu.SemaphoreType.DMA,
    ],
)(_sc_gather)


def _mlp_body(h_ref, lin_ref, w1_ref, b1_ref, g1_ref, e1_ref,
              w2_ref, b2_ref, g2_ref, e2_ref,
              w3_ref, b3_ref, g3_ref, e3_ref,
              wo_ref, bo_ref, out_ref):
    h = h_ref[...]
    z = jnp.dot(h, w1_ref[...], preferred_element_type=jnp.float32)
    z = (z + b1_ref[...]) * (g1_ref[...] * _BN_C) + e1_ref[...]
    a = jnp.maximum(z, 0.0)
    z = jnp.dot(a, w2_ref[...], preferred_element_type=jnp.float32)
    z = (z + b2_ref[...]) * (g2_ref[...] * _BN_C) + e2_ref[...]
    a = jnp.maximum(z, 0.0)
    z = jnp.dot(a, w3_ref[...], preferred_element_type=jnp.float32)
    z = (z + b3_ref[...]) * (g3_ref[...] * _BN_C) + e3_ref[...]
    a = jnp.maximum(z, 0.0)
    o = jnp.dot(a, wo_ref[...], preferred_element_type=jnp.float32)
    o = o + bo_ref[...] + lin_ref[...]
    out_ref[...] = 1.0 / (1.0 + jnp.exp(-o))


_BT = 512


def _mlp_call(h, lin2d, W1, b1, g1, be1, W2, b2, g2, be2,
              W3, b3, g3, be3, Wo, bo):
    full = lambda shape: pl.BlockSpec(shape, lambda i: (0, 0))
    return pl.pallas_call(
        _mlp_body,
        grid=(_BATCH // _BT,),
        in_specs=[
            pl.BlockSpec((_BT, 1920), lambda i: (i, 0)),
            pl.BlockSpec((_BT, 1), lambda i: (i, 0)),
            full((1920, 512)), full((1, 512)), full((1, 512)), full((1, 512)),
            full((512, 256)), full((1, 256)), full((1, 256)), full((1, 256)),
            full((256, 128)), full((1, 128)), full((1, 128)), full((1, 128)),
            full((128, 1)), full((1, 1)),
        ],
        out_specs=pl.BlockSpec((_BT, 1), lambda i: (i, 0)),
        out_shape=jax.ShapeDtypeStruct((_BATCH, 1), jnp.float32),
    )(h, lin2d, W1, b1, g1, be1, W2, b2, g2, be2,
      W3, b3, g3, be3, Wo, bo)


def kernel(x, additional, linear_w, linear_b, emb,
           W1, b1, g1, be1, W2, b2, g2, be2, W3, b3, g3, be3, Wo, bo):
    del additional
    xi = (x[:, _KEPT_COLS].astype(jnp.int32)
          + jnp.asarray(_OFFSETS)[None, :])
    idx2d = xi.reshape(_NIDX // _CHUNK, _CHUNK)

    rows, lin = _sc_gather_call(idx2d, emb, linear_w.reshape(_TABLE))
    h = rows.reshape(_BATCH, _NFIELD * _EMBED)
    lin2d = lin.reshape(_BATCH, 1) + linear_b[0]

    out = _mlp_call(h, lin2d,
                    W1, b1.reshape(1, -1), g1.reshape(1, -1),
                    be1.reshape(1, -1),
                    W2, b2.reshape(1, -1), g2.reshape(1, -1),
                    be2.reshape(1, -1),
                    W3, b3.reshape(1, -1), g3.reshape(1, -1),
                    be3.reshape(1, -1),
                    Wo, bo.reshape(1, 1))
    return out.reshape(_BATCH)


# speedup vs baseline: 9.9958x; 9.9958x over previous
import functools

import jax
import jax.numpy as jnp
import numpy as np
from jax import lax
from jax.experimental import pallas as pl
from jax.experimental.pallas import tpu as pltpu
from jax.experimental.pallas import tpu_sc as plsc

_BATCH = 4096
_NFIELD = 30
_EMBED = 64
_TABLE = 30000
_KEPT_COLS = np.array(
    [0, 1, 2, 4, 5, 6, 7, 10, 11, 12, 13, 14, 17, 18, 21, 22, 23]
    + list(range(26, 39)),
    dtype=np.int32,
)
_OFFSETS = (np.arange(_NFIELD, dtype=np.int32) * 1000)

_NC, _NS = 2, 16
_NW = _NC * _NS
_NIDX = _BATCH * _NFIELD
_IDX_W = _NIDX // _NW
_CHUNK = 128
_NCHUNK = _IDX_W // _CHUNK
_SAMP_W = _BATCH // _NW

_BN_C = float(1.0 / np.sqrt(1.0 + 1e-5))


def _sc_gather(idx2d_hbm, idxt_hbm, emb_hbm, linw_hbm, rows_out, lin_out,
               idx_v, idxt_v, rows_v, lin_v, linw_v, sem):
    wid = lax.axis_index("s") * _NC + lax.axis_index("c")
    pltpu.sync_copy(idx2d_hbm.at[wid], idx_v)
    pltpu.sync_copy(idxt_hbm.at[wid], idxt_v)
    pltpu.sync_copy(linw_hbm, linw_v)

    def chunk_body(c, carry):
        pltpu.async_copy(emb_hbm.at[idx_v.at[c]], rows_v, sem).wait()
        pltpu.sync_copy(
            rows_v, rows_out.at[pl.ds(wid * _IDX_W + c * _CHUNK, _CHUNK)])
        return carry

    lax.fori_loop(0, _NCHUNK, chunk_body, 0)

    for g in range(_SAMP_W // 16):
        lin_v[pl.ds(g * 16, 16)] = jnp.zeros((16,), jnp.float32)

    def lin_field(f, carry):
        def lin_group(g, carry2):
            idxs = idxt_v[pl.ds(f * _CHUNK + g * 16, 16)]
            vals = plsc.load_gather(linw_v, [idxs])
            lin_v[pl.ds(g * 16, 16)] = lin_v[pl.ds(g * 16, 16)] + vals
            return carry2

        return lax.fori_loop(0, _SAMP_W // 16, lin_group, carry)

    lax.fori_loop(0, _NFIELD, lin_field, 0)
    pltpu.sync_copy(lin_v, lin_out.at[pl.ds(wid * _SAMP_W, _SAMP_W)])


_sc_gather_call = functools.partial(
    pl.kernel,
    out_type=[
        jax.ShapeDtypeStruct((_NIDX, _EMBED), jnp.float32),
        jax.ShapeDtypeStruct((_BATCH,), jnp.float32),
    ],
    mesh=plsc.VectorSubcoreMesh(
        core_axis_name="c", subcore_axis_name="s",
        num_cores=_NC, num_subcores=_NS),
    compiler_params=pltpu.CompilerParams(
        use_tc_tiling_on_sc=False, needs_layout_passes=False),
    scratch_types=[
        pltpu.VMEM((_NCHUNK, _CHUNK), jnp.int32),
        pltpu.VMEM((_IDX_W,), jnp.int32),
        pltpu.VMEM((_CHUNK, _EMBED), jnp.float32),
        pltpu.VMEM((_SAMP_W,), jnp.float32),
        pltpu.VMEM((_TABLE,), jnp.float32),
        pltpu.SemaphoreType.DMA,
    ],
)(_sc_gather)


def _mlp_body(h_ref, lin_ref, w1_ref, b1_ref, g1_ref, e1_ref,
              w2_ref, b2_ref, g2_ref, e2_ref,
              w3_ref, b3_ref, g3_ref, e3_ref,
              wo_ref, bo_ref, out_ref):
    h = h_ref[...]
    z = jnp.dot(h, w1_ref[...], preferred_element_type=jnp.float32)
    z = (z + b1_ref[...]) * (g1_ref[...] * _BN_C) + e1_ref[...]
    a = jnp.maximum(z, 0.0)
    z = jnp.dot(a, w2_ref[...], preferred_element_type=jnp.float32)
    z = (z + b2_ref[...]) * (g2_ref[...] * _BN_C) + e2_ref[...]
    a = jnp.maximum(z, 0.0)
    z = jnp.dot(a, w3_ref[...], preferred_element_type=jnp.float32)
    z = (z + b3_ref[...]) * (g3_ref[...] * _BN_C) + e3_ref[...]
    a = jnp.maximum(z, 0.0)
    o = jnp.dot(a, wo_ref[...], preferred_element_type=jnp.float32)
    o = o + bo_ref[...] + lin_ref[...]
    out_ref[...] = 1.0 / (1.0 + jnp.exp(-o))


_BT = 512


def _mlp_call(h, lin2d, W1, b1, g1, be1, W2, b2, g2, be2,
              W3, b3, g3, be3, Wo, bo):
    full = lambda shape: pl.BlockSpec(shape, lambda i: (0, 0))
    return pl.pallas_call(
        _mlp_body,
        grid=(_BATCH // _BT,),
        in_specs=[
            pl.BlockSpec((_BT, 1920), lambda i: (i, 0)),
            pl.BlockSpec((_BT, 1), lambda i: (i, 0)),
            full((1920, 512)), full((1, 512)), full((1, 512)), full((1, 512)),
            full((512, 256)), full((1, 256)), full((1, 256)), full((1, 256)),
            full((256, 128)), full((1, 128)), full((1, 128)), full((1, 128)),
            full((128, 1)), full((1, 1)),
        ],
        out_specs=pl.BlockSpec((_BT, 1), lambda i: (i, 0)),
        out_shape=jax.ShapeDtypeStruct((_BATCH, 1), jnp.float32),
    )(h, lin2d, W1, b1, g1, be1, W2, b2, g2, be2,
      W3, b3, g3, be3, Wo, bo)


def kernel(x, additional, linear_w, linear_b, emb,
           W1, b1, g1, be1, W2, b2, g2, be2, W3, b3, g3, be3, Wo, bo):
    del additional
    xi = (x[:, _KEPT_COLS].astype(jnp.int32)
          + jnp.asarray(_OFFSETS)[None, :])
    idx2d = xi.reshape(_NW, _NCHUNK, _CHUNK)
    idxt = (xi.reshape(_NW, _SAMP_W, _NFIELD)
            .transpose(0, 2, 1).reshape(_NW, _IDX_W))

    rows, lin = _sc_gather_call(idx2d, idxt, emb, linear_w.reshape(_TABLE))
    h = rows.reshape(_BATCH, _NFIELD * _EMBED)
    lin2d = lin.reshape(_BATCH, 1) + linear_b[0]

    out = _mlp_call(h, lin2d,
                    W1, b1.reshape(1, -1), g1.reshape(1, -1),
                    be1.reshape(1, -1),
                    W2, b2.reshape(1, -1), g2.reshape(1, -1),
                    be2.reshape(1, -1),
                    W3, b3.reshape(1, -1), g3.reshape(1, -1),
                    be3.reshape(1, -1),
                    Wo, bo.reshape(1, 1))
    return out.reshape(_BATCH)


# speedup vs baseline: 11.2093x; 1.1214x over previous
import functools

import jax
import jax.numpy as jnp
import numpy as np
from jax import lax
from jax.experimental import pallas as pl
from jax.experimental.pallas import tpu as pltpu
from jax.experimental.pallas import tpu_sc as plsc

_BATCH = 4096
_NFIELD = 30
_EMBED = 64
_TABLE = 30000
_KEPT_COLS = np.array(
    [0, 1, 2, 4, 5, 6, 7, 10, 11, 12, 13, 14, 17, 18, 21, 22, 23]
    + list(range(26, 39)),
    dtype=np.int32,
)
_OFFSETS = (np.arange(_NFIELD, dtype=np.int32) * 1000)

_NC, _NS = 2, 16
_NW = _NC * _NS
_NIDX = _BATCH * _NFIELD
_IDX_W = _NIDX // _NW
_CHUNK = 128
_NCHUNK = _IDX_W // _CHUNK
_SAMP_W = _BATCH // _NW

_BN_C = float(1.0 / np.sqrt(1.0 + 1e-5))


def _sc_gather(idx2d_hbm, idxt_hbm, emb_hbm, linw_hbm, rows_out, lin_out,
               idx_v, idxt_v, rows_v, lin_v, linw_v, sem0, sem1):
    wid = lax.axis_index("s") * _NC + lax.axis_index("c")
    pltpu.sync_copy(idx2d_hbm.at[wid], idx_v)
    pltpu.sync_copy(idxt_hbm.at[wid], idxt_v)
    pltpu.sync_copy(linw_hbm, linw_v)

    out_base = wid * _IDX_W
    sems = (sem0, sem1)

    def _start(c, b):
        pltpu.async_copy(emb_hbm.at[idx_v.at[c]], rows_v.at[b], sems[b])

    def _finish(c, b):
        pltpu.make_async_copy(
            emb_hbm.at[idx_v.at[c]], rows_v.at[b], sems[b]).wait()
        pltpu.sync_copy(
            rows_v.at[b], rows_out.at[pl.ds(out_base + c * _CHUNK, _CHUNK)])

    _start(0, 0)

    def chunk_body(c0, carry):
        @pl.when(c0 + 1 < _NCHUNK)
        def _():
            _start(c0 + 1, 1)
        _finish(c0, 0)

        @pl.when(c0 + 2 < _NCHUNK)
        def _():
            _start(c0 + 2, 0)

        @pl.when(c0 + 1 < _NCHUNK)
        def _():
            _finish(c0 + 1, 1)
        return carry

    lax.fori_loop(0, _NCHUNK // 2, lambda i, c: chunk_body(2 * i, c), 0)

    for g in range(_SAMP_W // 16):
        lin_v[pl.ds(g * 16, 16)] = jnp.zeros((16,), jnp.float32)

    def lin_field(f, carry):
        def lin_group(g, carry2):
            idxs = idxt_v[pl.ds(f * _CHUNK + g * 16, 16)]
            vals = plsc.load_gather(linw_v, [idxs])
            lin_v[pl.ds(g * 16, 16)] = lin_v[pl.ds(g * 16, 16)] + vals
            return carry2

        return lax.fori_loop(0, _SAMP_W // 16, lin_group, carry)

    lax.fori_loop(0, _NFIELD, lin_field, 0)
    pltpu.sync_copy(lin_v, lin_out.at[pl.ds(wid * _SAMP_W, _SAMP_W)])


_sc_gather_call = functools.partial(
    pl.kernel,
    out_type=[
        jax.ShapeDtypeStruct((_NIDX, _EMBED), jnp.float32),
        jax.ShapeDtypeStruct((_BATCH,), jnp.float32),
    ],
    mesh=plsc.VectorSubcoreMesh(
        core_axis_name="c", subcore_axis_name="s",
        num_cores=_NC, num_subcores=_NS),
    compiler_params=pltpu.CompilerParams(
        use_tc_tiling_on_sc=False, needs_layout_passes=False),
    scratch_types=[
        pltpu.VMEM((_NCHUNK, _CHUNK), jnp.int32),
        pltpu.VMEM((_IDX_W,), jnp.int32),
        pltpu.VMEM((2, _CHUNK, _EMBED), jnp.float32),
        pltpu.VMEM((_SAMP_W,), jnp.float32),
        pltpu.VMEM((_TABLE,), jnp.float32),
        pltpu.SemaphoreType.DMA,
        pltpu.SemaphoreType.DMA,
    ],
)(_sc_gather)


def _mlp_body(h_ref, lin_ref, w1_ref, b1_ref, g1_ref, e1_ref,
              w2_ref, b2_ref, g2_ref, e2_ref,
              w3_ref, b3_ref, g3_ref, e3_ref,
              wo_ref, bo_ref, out_ref):
    h = h_ref[...]
    z = jnp.dot(h, w1_ref[...], preferred_element_type=jnp.float32)
    z = (z + b1_ref[...]) * (g1_ref[...] * _BN_C) + e1_ref[...]
    a = jnp.maximum(z, 0.0)
    z = jnp.dot(a, w2_ref[...], preferred_element_type=jnp.float32)
    z = (z + b2_ref[...]) * (g2_ref[...] * _BN_C) + e2_ref[...]
    a = jnp.maximum(z, 0.0)
    z = jnp.dot(a, w3_ref[...], preferred_element_type=jnp.float32)
    z = (z + b3_ref[...]) * (g3_ref[...] * _BN_C) + e3_ref[...]
    a = jnp.maximum(z, 0.0)
    o = jnp.dot(a, wo_ref[...], preferred_element_type=jnp.float32)
    o = o + bo_ref[...] + lin_ref[...]
    out_ref[...] = 1.0 / (1.0 + jnp.exp(-o))


_BT = 512


def _mlp_call(h, lin2d, W1, b1, g1, be1, W2, b2, g2, be2,
              W3, b3, g3, be3, Wo, bo):
    full = lambda shape: pl.BlockSpec(shape, lambda i: (0, 0))
    return pl.pallas_call(
        _mlp_body,
        grid=(_BATCH // _BT,),
        in_specs=[
            pl.BlockSpec((_BT, 1920), lambda i: (i, 0)),
            pl.BlockSpec((_BT, 1), lambda i: (i, 0)),
            full((1920, 512)), full((1, 512)), full((1, 512)), full((1, 512)),
            full((512, 256)), full((1, 256)), full((1, 256)), full((1, 256)),
            full((256, 128)), full((1, 128)), full((1, 128)), full((1, 128)),
            full((128, 1)), full((1, 1)),
        ],
        out_specs=pl.BlockSpec((_BT, 1), lambda i: (i, 0)),
        out_shape=jax.ShapeDtypeStruct((_BATCH, 1), jnp.float32),
    )(h, lin2d, W1, b1, g1, be1, W2, b2, g2, be2,
      W3, b3, g3, be3, Wo, bo)


def kernel(x, additional, linear_w, linear_b, emb,
           W1, b1, g1, be1, W2, b2, g2, be2, W3, b3, g3, be3, Wo, bo):
    del additional
    xi = (x[:, _KEPT_COLS].astype(jnp.int32)
          + jnp.asarray(_OFFSETS)[None, :])
    idx2d = xi.reshape(_NW, _NCHUNK, _CHUNK)
    idxt = (xi.reshape(_NW, _SAMP_W, _NFIELD)
            .transpose(0, 2, 1).reshape(_NW, _IDX_W))

    rows, lin = _sc_gather_call(idx2d, idxt, emb, linear_w.reshape(_TABLE))
    h = rows.reshape(_BATCH, _NFIELD * _EMBED)
    lin2d = lin.reshape(_BATCH, 1) + linear_b[0]

    out = _mlp_call(h, lin2d,
                    W1, b1.reshape(1, -1), g1.reshape(1, -1),
                    be1.reshape(1, -1),
                    W2, b2.reshape(1, -1), g2.reshape(1, -1),
                    be2.reshape(1, -1),
                    W3, b3.reshape(1, -1), g3.reshape(1, -1),
                    be3.reshape(1, -1),
                    Wo, bo.reshape(1, 1))
    return out.reshape(_BATCH)


# speedup vs baseline: 14.4710x; 1.2910x over previous
import functools

import jax
import jax.numpy as jnp
import numpy as np
from jax import lax
from jax.experimental import pallas as pl
from jax.experimental.pallas import tpu as pltpu
from jax.experimental.pallas import tpu_sc as plsc

_BATCH = 4096
_NFIELD = 30
_EMBED = 64
_TABLE = 30000
_KEPT_COLS = np.array(
    [0, 1, 2, 4, 5, 6, 7, 10, 11, 12, 13, 14, 17, 18, 21, 22, 23]
    + list(range(26, 39)),
    dtype=np.int32,
)
_OFFSETS = (np.arange(_NFIELD, dtype=np.int32) * 1000)

_NC, _NS = 2, 16
_NW = _NC * _NS
_NIDX = _BATCH * _NFIELD
_IDX_W = _NIDX // _NW
_CHUNK = 128
_NCHUNK = _IDX_W // _CHUNK
_SAMP_W = _BATCH // _NW

_BN_C = float(1.0 / np.sqrt(1.0 + 1e-5))


def _sc_gather(idx2d_hbm, idxt_hbm, emb_hbm, linw_hbm, rows_out, lin_out,
               idx_v, idxt_v, rows_v, lin_v, linw_v, sem0, sem1):
    wid = lax.axis_index("s") * _NC + lax.axis_index("c")
    pltpu.sync_copy(idx2d_hbm.at[wid], idx_v)
    pltpu.sync_copy(idxt_hbm.at[wid], idxt_v)
    pltpu.sync_copy(linw_hbm, linw_v)

    out_base = wid * (_IDX_W // 2)
    sems = (sem0, sem1)

    half = _CHUNK // 2

    def _start(c, b):
        pltpu.async_copy(emb_hbm.at[idx_v.at[c, 0]], rows_v.at[b, 0],
                         sems[b])
        pltpu.async_copy(emb_hbm.at[idx_v.at[c, 1]], rows_v.at[b, 1],
                         sems[b])

    def _finish(c, b):
        pltpu.make_async_copy(
            emb_hbm.at[idx_v.at[c, 0]], rows_v.at[b, 0], sems[b]).wait()
        pltpu.make_async_copy(
            emb_hbm.at[idx_v.at[c, 1]], rows_v.at[b, 1], sems[b]).wait()
        q0 = out_base + c * half
        k, s0 = q0 // _BATCH, q0 % _BATCH
        pltpu.sync_copy(
            rows_v.at[b, 0],
            rows_out.at[k, pl.ds(s0, half), pl.ds(0, _EMBED)])
        pltpu.sync_copy(
            rows_v.at[b, 1],
            rows_out.at[k, pl.ds(s0, half), pl.ds(_EMBED, _EMBED)])

    _start(0, 0)

    def chunk_body(c0, carry):
        @pl.when(c0 + 1 < _NCHUNK)
        def _():
            _start(c0 + 1, 1)
        _finish(c0, 0)

        @pl.when(c0 + 2 < _NCHUNK)
        def _():
            _start(c0 + 2, 0)

        @pl.when(c0 + 1 < _NCHUNK)
        def _():
            _finish(c0 + 1, 1)
        return carry

    lax.fori_loop(0, _NCHUNK // 2, lambda i, c: chunk_body(2 * i, c), 0)

    for g in range(_SAMP_W // 16):
        lin_v[pl.ds(g * 16, 16)] = jnp.zeros((16,), jnp.float32)

    def lin_field(f, carry):
        def lin_group(g, carry2):
            idxs = idxt_v[pl.ds(f * _CHUNK + g * 16, 16)]
            vals = plsc.load_gather(linw_v, [idxs])
            lin_v[pl.ds(g * 16, 16)] = lin_v[pl.ds(g * 16, 16)] + vals
            return carry2

        return lax.fori_loop(0, _SAMP_W // 16, lin_group, carry)

    lax.fori_loop(0, _NFIELD, lin_field, 0)
    pltpu.sync_copy(lin_v, lin_out.at[pl.ds(wid * _SAMP_W, _SAMP_W)])


_sc_gather_call = functools.partial(
    pl.kernel,
    out_type=[
        jax.ShapeDtypeStruct((_NFIELD // 2, _BATCH, 2 * _EMBED), jnp.float32),
        jax.ShapeDtypeStruct((_BATCH,), jnp.float32),
    ],
    mesh=plsc.VectorSubcoreMesh(
        core_axis_name="c", subcore_axis_name="s",
        num_cores=_NC, num_subcores=_NS),
    compiler_params=pltpu.CompilerParams(
        use_tc_tiling_on_sc=False, needs_layout_passes=False),
    scratch_types=[
        pltpu.VMEM((_NCHUNK, 2, _CHUNK // 2), jnp.int32),
        pltpu.VMEM((_IDX_W,), jnp.int32),
        pltpu.VMEM((2, 2, _CHUNK // 2, _EMBED), jnp.float32),
        pltpu.VMEM((_SAMP_W,), jnp.float32),
        pltpu.VMEM((_TABLE,), jnp.float32),
        pltpu.SemaphoreType.DMA,
        pltpu.SemaphoreType.DMA,
    ],
)(_sc_gather)


def _mlp_body(h_ref, lin_ref, w1_ref, b1_ref, g1_ref, e1_ref,
              w2_ref, b2_ref, g2_ref, e2_ref,
              w3_ref, b3_ref, g3_ref, e3_ref,
              wo_ref, bo_ref, out_ref):
    h = jnp.concatenate(
        [h_ref[k] for k in range(_NFIELD // 2)], axis=1)
    z = jnp.dot(h, w1_ref[...], preferred_element_type=jnp.float32)
    z = (z + b1_ref[...]) * (g1_ref[...] * _BN_C) + e1_ref[...]
    a = jnp.maximum(z, 0.0)
    z = jnp.dot(a, w2_ref[...], preferred_element_type=jnp.float32)
    z = (z + b2_ref[...]) * (g2_ref[...] * _BN_C) + e2_ref[...]
    a = jnp.maximum(z, 0.0)
    z = jnp.dot(a, w3_ref[...], preferred_element_type=jnp.float32)
    z = (z + b3_ref[...]) * (g3_ref[...] * _BN_C) + e3_ref[...]
    a = jnp.maximum(z, 0.0)
    o = jnp.dot(a, wo_ref[...], preferred_element_type=jnp.float32)
    o = o + bo_ref[...] + lin_ref[...]
    out_ref[...] = 1.0 / (1.0 + jnp.exp(-o))


_BT = 512


def _mlp_call(h, lin2d, W1, b1, g1, be1, W2, b2, g2, be2,
              W3, b3, g3, be3, Wo, bo):
    full = lambda shape: pl.BlockSpec(shape, lambda i: (0, 0))
    return pl.pallas_call(
        _mlp_body,
        grid=(_BATCH // _BT,),
        in_specs=[
            pl.BlockSpec((_NFIELD // 2, _BT, 128), lambda i: (0, i, 0)),
            pl.BlockSpec((_BT, 1), lambda i: (i, 0)),
            full((1920, 512)), full((1, 512)), full((1, 512)), full((1, 512)),
            full((512, 256)), full((1, 256)), full((1, 256)), full((1, 256)),
            full((256, 128)), full((1, 128)), full((1, 128)), full((1, 128)),
            full((128, 1)), full((1, 1)),
        ],
        out_specs=pl.BlockSpec((_BT, 1), lambda i: (i, 0)),
        out_shape=jax.ShapeDtypeStruct((_BATCH, 1), jnp.float32),
    )(h, lin2d, W1, b1, g1, be1, W2, b2, g2, be2,
      W3, b3, g3, be3, Wo, bo)


def kernel(x, additional, linear_w, linear_b, emb,
           W1, b1, g1, be1, W2, b2, g2, be2, W3, b3, g3, be3, Wo, bo):
    del additional
    xi = (x[:, _KEPT_COLS].astype(jnp.int32)
          + jnp.asarray(_OFFSETS)[None, :])
    ev = xi[:, 0::2].T.reshape(_NW, _NCHUNK, _CHUNK // 2)
    od = xi[:, 1::2].T.reshape(_NW, _NCHUNK, _CHUNK // 2)
    idxq = jnp.stack([ev, od], axis=2)
    idxt = (xi.reshape(_NW, _SAMP_W, _NFIELD)
            .transpose(0, 2, 1).reshape(_NW, _IDX_W))

    rows, lin = _sc_gather_call(idxq, idxt, emb, linear_w.reshape(_TABLE))
    lin2d = lin.reshape(_BATCH, 1) + linear_b[0]

    out = _mlp_call(rows, lin2d,
                    W1, b1.reshape(1, -1), g1.reshape(1, -1),
                    be1.reshape(1, -1),
                    W2, b2.reshape(1, -1), g2.reshape(1, -1),
                    be2.reshape(1, -1),
                    W3, b3.reshape(1, -1), g3.reshape(1, -1),
                    be3.reshape(1, -1),
                    Wo, bo.reshape(1, 1))
    return out.reshape(_BATCH)


# speedup vs baseline: 15.2145x; 1.0514x over previous
import functools

import jax
import jax.numpy as jnp
import numpy as np
from jax import lax
from jax.experimental import pallas as pl
from jax.experimental.pallas import tpu as pltpu
from jax.experimental.pallas import tpu_sc as plsc

_BATCH = 4096
_NFIELD = 30
_EMBED = 64
_TABLE = 30000
_KEPT_COLS = np.array(
    [0, 1, 2, 4, 5, 6, 7, 10, 11, 12, 13, 14, 17, 18, 21, 22, 23]
    + list(range(26, 39)),
    dtype=np.int32,
)
_OFFSETS = (np.arange(_NFIELD, dtype=np.int32) * 1000)

_NC, _NS = 2, 16
_NW = _NC * _NS
_NIDX = _BATCH * _NFIELD
_IDX_W = _NIDX // _NW
_CHUNK = 128
_NCHUNK = _IDX_W // _CHUNK
_SAMP_W = _BATCH // _NW

_BN_C = float(1.0 / np.sqrt(1.0 + 1e-5))


def _sc_gather(idx2d_hbm, idxt_hbm, emb_hbm, linw_hbm, rows_out, lin_out,
               idx_v, idxt_v, rows_v, lin_v, linw_v,
               g0, g1, g2, g3, w0, w1, w2, w3):
    gsems = (g0, g1, g2, g3)
    wsems = (w0, w1, w2, w3)
    wid = lax.axis_index("s") * _NC + lax.axis_index("c")
    pltpu.sync_copy(idx2d_hbm.at[wid], idx_v)
    pltpu.sync_copy(idxt_hbm.at[wid], idxt_v)
    pltpu.sync_copy(linw_hbm, linw_v)

    out_base = wid * (_IDX_W // 2)
    half = _CHUNK // 2
    _NB = 4

    def _start_gather(c, b):
        pltpu.async_copy(emb_hbm.at[idx_v.at[c, 0]], rows_v.at[b, 0],
                         gsems[b])
        pltpu.async_copy(emb_hbm.at[idx_v.at[c, 1]], rows_v.at[b, 1],
                         gsems[b])

    def _wait_gather(c, b):
        pltpu.make_async_copy(
            emb_hbm.at[idx_v.at[c, 0]], rows_v.at[b, 0], gsems[b]).wait()
        pltpu.make_async_copy(
            emb_hbm.at[idx_v.at[c, 1]], rows_v.at[b, 1], gsems[b]).wait()

    def _wb_descr(c, b):
        q0 = out_base + c * half
        k, s0 = q0 // _BATCH, q0 % _BATCH
        return (
            pltpu.make_async_copy(
                rows_v.at[b, 0],
                rows_out.at[k, pl.ds(s0, half), pl.ds(0, _EMBED)],
                wsems[b]),
            pltpu.make_async_copy(
                rows_v.at[b, 1],
                rows_out.at[k, pl.ds(s0, half), pl.ds(_EMBED, _EMBED)],
                wsems[b]),
        )

    def _start_wb(c, b):
        for d in _wb_descr(c, b):
            d.start()

    def _wait_wb(c, b):
        for d in _wb_descr(c, b):
            d.wait()

    for c in range(_NB - 1):
        _start_gather(c, c)

    def ring_body(i, carry):
        for b in range(_NB):
            c = _NB * i + b

            @pl.when(c < _NCHUNK)
            def _():
                _wait_gather(c, b)
                _start_wb(c, b)

            cn = c + _NB - 1
            bn = (b + _NB - 1) % _NB

            @pl.when(cn < _NCHUNK)
            def _():
                @pl.when(cn >= _NB)
                def _():
                    _wait_wb(cn - _NB, bn)
                _start_gather(cn, bn)
        return carry

    lax.fori_loop(0, (_NCHUNK + _NB - 1) // _NB, ring_body, 0)
    for c in range(_NCHUNK - _NB, _NCHUNK):
        _wait_wb(c, c % _NB)

    for g in range(_SAMP_W // 16):
        lin_v[pl.ds(g * 16, 16)] = jnp.zeros((16,), jnp.float32)

    def lin_field(f, carry):
        def lin_group(g, carry2):
            idxs = idxt_v[pl.ds(f * _CHUNK + g * 16, 16)]
            vals = plsc.load_gather(linw_v, [idxs])
            lin_v[pl.ds(g * 16, 16)] = lin_v[pl.ds(g * 16, 16)] + vals
            return carry2

        return lax.fori_loop(0, _SAMP_W // 16, lin_group, carry)

    lax.fori_loop(0, _NFIELD, lin_field, 0)
    pltpu.sync_copy(lin_v, lin_out.at[pl.ds(wid * _SAMP_W, _SAMP_W)])


_sc_gather_call = functools.partial(
    pl.kernel,
    out_type=[
        jax.ShapeDtypeStruct((_NFIELD // 2, _BATCH, 2 * _EMBED), jnp.float32),
        jax.ShapeDtypeStruct((_BATCH,), jnp.float32),
    ],
    mesh=plsc.VectorSubcoreMesh(
        core_axis_name="c", subcore_axis_name="s",
        num_cores=_NC, num_subcores=_NS),
    compiler_params=pltpu.CompilerParams(
        use_tc_tiling_on_sc=False, needs_layout_passes=False),
    scratch_types=[
        pltpu.VMEM((_NCHUNK, 2, _CHUNK // 2), jnp.int32),
        pltpu.VMEM((_IDX_W,), jnp.int32),
        pltpu.VMEM((4, 2, _CHUNK // 2, _EMBED), jnp.float32),
        pltpu.VMEM((_SAMP_W,), jnp.float32),
        pltpu.VMEM((_TABLE,), jnp.float32),
        pltpu.SemaphoreType.DMA, pltpu.SemaphoreType.DMA,
        pltpu.SemaphoreType.DMA, pltpu.SemaphoreType.DMA,
        pltpu.SemaphoreType.DMA, pltpu.SemaphoreType.DMA,
        pltpu.SemaphoreType.DMA, pltpu.SemaphoreType.DMA,
    ],
)(_sc_gather)


def _mlp_body(h_ref, lin_ref, w1_ref, b1_ref, g1_ref, e1_ref,
              w2_ref, b2_ref, g2_ref, e2_ref,
              w3_ref, b3_ref, g3_ref, e3_ref,
              wo_ref, bo_ref, out_ref):
    h = jnp.concatenate(
        [h_ref[k] for k in range(_NFIELD // 2)], axis=1)
    z = jnp.dot(h, w1_ref[...], preferred_element_type=jnp.float32)
    z = (z + b1_ref[...]) * (g1_ref[...] * _BN_C) + e1_ref[...]
    a = jnp.maximum(z, 0.0)
    z = jnp.dot(a, w2_ref[...], preferred_element_type=jnp.float32)
    z = (z + b2_ref[...]) * (g2_ref[...] * _BN_C) + e2_ref[...]
    a = jnp.maximum(z, 0.0)
    z = jnp.dot(a, w3_ref[...], preferred_element_type=jnp.float32)
    z = (z + b3_ref[...]) * (g3_ref[...] * _BN_C) + e3_ref[...]
    a = jnp.maximum(z, 0.0)
    o = jnp.dot(a, wo_ref[...], preferred_element_type=jnp.float32)
    o = o + bo_ref[...] + lin_ref[...]
    out_ref[...] = 1.0 / (1.0 + jnp.exp(-o))


_BT = 512


def _mlp_call(h, lin2d, W1, b1, g1, be1, W2, b2, g2, be2,
              W3, b3, g3, be3, Wo, bo):
    full = lambda shape: pl.BlockSpec(shape, lambda i: (0, 0))
    return pl.pallas_call(
        _mlp_body,
        grid=(_BATCH // _BT,),
        in_specs=[
            pl.BlockSpec((_NFIELD // 2, _BT, 128), lambda i: (0, i, 0)),
            pl.BlockSpec((_BT, 1), lambda i: (i, 0)),
            full((1920, 512)), full((1, 512)), full((1, 512)), full((1, 512)),
            full((512, 256)), full((1, 256)), full((1, 256)), full((1, 256)),
            full((256, 128)), full((1, 128)), full((1, 128)), full((1, 128)),
            full((128, 1)), full((1, 1)),
        ],
        out_specs=pl.BlockSpec((_BT, 1), lambda i: (i, 0)),
        out_shape=jax.ShapeDtypeStruct((_BATCH, 1), jnp.float32),
    )(h, lin2d, W1, b1, g1, be1, W2, b2, g2, be2,
      W3, b3, g3, be3, Wo, bo)


def kernel(x, additional, linear_w, linear_b, emb,
           W1, b1, g1, be1, W2, b2, g2, be2, W3, b3, g3, be3, Wo, bo):
    del additional
    xi = (x[:, _KEPT_COLS].astype(jnp.int32)
          + jnp.asarray(_OFFSETS)[None, :])
    ev = xi[:, 0::2].T.reshape(_NW, _NCHUNK, _CHUNK // 2)
    od = xi[:, 1::2].T.reshape(_NW, _NCHUNK, _CHUNK // 2)
    idxq = jnp.stack([ev, od], axis=2)
    idxt = (xi.reshape(_NW, _SAMP_W, _NFIELD)
            .transpose(0, 2, 1).reshape(_NW, _IDX_W))

    rows, lin = _sc_gather_call(idxq, idxt, emb, linear_w.reshape(_TABLE))
    lin2d = lin.reshape(_BATCH, 1) + linear_b[0]

    out = _mlp_call(rows, lin2d,
                    W1, b1.reshape(1, -1), g1.reshape(1, -1),
                    be1.reshape(1, -1),
                    W2, b2.reshape(1, -1), g2.reshape(1, -1),
                    be2.reshape(1, -1),
                    W3, b3.reshape(1, -1), g3.reshape(1, -1),
                    be3.reshape(1, -1),
                    Wo, bo.reshape(1, 1))
    return out.reshape(_BATCH)


# speedup vs baseline: 15.2686x; 1.0036x over previous
import functools

import jax
import jax.numpy as jnp
import numpy as np
from jax import lax
from jax.experimental import pallas as pl
from jax.experimental.pallas import tpu as pltpu
from jax.experimental.pallas import tpu_sc as plsc

_BATCH = 4096
_NFIELD = 30
_EMBED = 64
_TABLE = 30000
_KEPT_COLS = np.array(
    [0, 1, 2, 4, 5, 6, 7, 10, 11, 12, 13, 14, 17, 18, 21, 22, 23]
    + list(range(26, 39)),
    dtype=np.int32,
)
_OFFSETS = (np.arange(_NFIELD, dtype=np.int32) * 1000)

_NC, _NS = 2, 16
_NW = _NC * _NS
_NIDX = _BATCH * _NFIELD
_IDX_W = _NIDX // _NW
_CHUNK = 128
_NCHUNK = _IDX_W // _CHUNK
_SAMP_W = _BATCH // _NW

_BN_C = float(1.0 / np.sqrt(1.0 + 1e-5))


def _sc_gather(idx2d_hbm, idxt_hbm, emb_hbm, linw_hbm, rows_out, lin_out,
               idx_v, idxt_v, rows_v, lin_v, linw_v,
               g0, g1, g2, g3, w0, w1, w2, w3):
    gsems = (g0, g1, g2, g3)
    wsems = (w0, w1, w2, w3)
    wid = lax.axis_index("s") * _NC + lax.axis_index("c")
    pltpu.sync_copy(idx2d_hbm.at[wid], idx_v)
    pltpu.sync_copy(idxt_hbm.at[wid], idxt_v)
    pltpu.sync_copy(linw_hbm, linw_v)

    out_base = wid * (_IDX_W // 2)
    half = _CHUNK // 2
    _NB = 4

    def _start_gather(c, b):
        pltpu.async_copy(emb_hbm.at[idx_v.at[c, 0]], rows_v.at[b, 0],
                         gsems[b])
        pltpu.async_copy(emb_hbm.at[idx_v.at[c, 1]], rows_v.at[b, 1],
                         gsems[b])

    def _wait_gather(c, b):
        pltpu.make_async_copy(
            emb_hbm.at[idx_v.at[c, 0]], rows_v.at[b, 0], gsems[b]).wait()
        pltpu.make_async_copy(
            emb_hbm.at[idx_v.at[c, 1]], rows_v.at[b, 1], gsems[b]).wait()

    def _wb_descr(c, b):
        q0 = out_base + c * half
        k, s0 = q0 // _BATCH, q0 % _BATCH
        return (
            pltpu.make_async_copy(
                rows_v.at[b, 0],
                rows_out.at[k, pl.ds(s0, half), pl.ds(0, _EMBED)],
                wsems[b]),
            pltpu.make_async_copy(
                rows_v.at[b, 1],
                rows_out.at[k, pl.ds(s0, half), pl.ds(_EMBED, _EMBED)],
                wsems[b]),
        )

    def _start_wb(c, b):
        for d in _wb_descr(c, b):
            d.start()

    def _wait_wb(c, b):
        for d in _wb_descr(c, b):
            d.wait()

    for c in range(_NB - 1):
        _start_gather(c, c)

    def ring_body(i, carry):
        for b in range(_NB):
            c = _NB * i + b

            @pl.when(c < _NCHUNK)
            def _():
                _wait_gather(c, b)
                _start_wb(c, b)

            cn = c + _NB - 1
            bn = (b + _NB - 1) % _NB

            @pl.when(cn < _NCHUNK)
            def _():
                @pl.when(cn >= _NB)
                def _():
                    _wait_wb(cn - _NB, bn)
                _start_gather(cn, bn)
        return carry

    lax.fori_loop(0, (_NCHUNK + _NB - 1) // _NB, ring_body, 0)
    for c in range(_NCHUNK - _NB, _NCHUNK):
        _wait_wb(c, c % _NB)

    for g in range(_SAMP_W // 16):
        lin_v[pl.ds(g * 16, 16)] = jnp.zeros((16,), jnp.float32)

    def lin_field(f, carry):
        def lin_group(g, carry2):
            idxs = idxt_v[pl.ds(f * _CHUNK + g * 16, 16)]
            vals = plsc.load_gather(linw_v, [idxs])
            lin_v[pl.ds(g * 16, 16)] = lin_v[pl.ds(g * 16, 16)] + vals
            return carry2

        return lax.fori_loop(0, _SAMP_W // 16, lin_group, carry)

    lax.fori_loop(0, _NFIELD, lin_field, 0)
    pltpu.sync_copy(lin_v, lin_out.at[pl.ds(wid * _SAMP_W, _SAMP_W)])


_sc_gather_call = functools.partial(
    pl.kernel,
    out_type=[
        jax.ShapeDtypeStruct((_NFIELD // 2, _BATCH, 2 * _EMBED), jnp.float32),
        jax.ShapeDtypeStruct((_BATCH,), jnp.float32),
    ],
    mesh=plsc.VectorSubcoreMesh(
        core_axis_name="c", subcore_axis_name="s",
        num_cores=_NC, num_subcores=_NS),
    compiler_params=pltpu.CompilerParams(
        use_tc_tiling_on_sc=False, needs_layout_passes=False),
    scratch_types=[
        pltpu.VMEM((_NCHUNK, 2, _CHUNK // 2), jnp.int32),
        pltpu.VMEM((_IDX_W,), jnp.int32),
        pltpu.VMEM((4, 2, _CHUNK // 2, _EMBED), jnp.float32),
        pltpu.VMEM((_SAMP_W,), jnp.float32),
        pltpu.VMEM((_TABLE,), jnp.float32),
        pltpu.SemaphoreType.DMA, pltpu.SemaphoreType.DMA,
        pltpu.SemaphoreType.DMA, pltpu.SemaphoreType.DMA,
        pltpu.SemaphoreType.DMA, pltpu.SemaphoreType.DMA,
        pltpu.SemaphoreType.DMA, pltpu.SemaphoreType.DMA,
    ],
)(_sc_gather)


def _mlp_body(h_ref, lin_ref, w1_ref, b1_ref, g1_ref, e1_ref,
              w2_ref, b2_ref, g2_ref, e2_ref,
              w3_ref, b3_ref, g3_ref, e3_ref,
              wo_ref, bo_ref, out_ref):
    h = jnp.concatenate(
        [h_ref[k] for k in range(_NFIELD // 2)], axis=1)
    z = jnp.dot(h.astype(jnp.bfloat16), w1_ref[...],
                preferred_element_type=jnp.float32)
    z = (z + b1_ref[...]) * (g1_ref[...] * _BN_C) + e1_ref[...]
    a = jnp.maximum(z, 0.0)
    z = jnp.dot(a.astype(jnp.bfloat16), w2_ref[...],
                preferred_element_type=jnp.float32)
    z = (z + b2_ref[...]) * (g2_ref[...] * _BN_C) + e2_ref[...]
    a = jnp.maximum(z, 0.0)
    z = jnp.dot(a.astype(jnp.bfloat16), w3_ref[...],
                preferred_element_type=jnp.float32)
    z = (z + b3_ref[...]) * (g3_ref[...] * _BN_C) + e3_ref[...]
    a = jnp.maximum(z, 0.0)
    o = jnp.dot(a.astype(jnp.bfloat16), wo_ref[...],
                preferred_element_type=jnp.float32)
    o = o + bo_ref[...] + lin_ref[...]
    out_ref[...] = 1.0 / (1.0 + jnp.exp(-o))


_BT = 512


def _mlp_call(h, lin2d, W1, b1, g1, be1, W2, b2, g2, be2,
              W3, b3, g3, be3, Wo, bo):
    full = lambda shape: pl.BlockSpec(shape, lambda i: (0, 0))
    return pl.pallas_call(
        _mlp_body,
        grid=(_BATCH // _BT,),
        in_specs=[
            pl.BlockSpec((_NFIELD // 2, _BT, 128), lambda i: (0, i, 0)),
            pl.BlockSpec((_BT, 1), lambda i: (i, 0)),
            full((1920, 512)), full((1, 512)), full((1, 512)), full((1, 512)),
            full((512, 256)), full((1, 256)), full((1, 256)), full((1, 256)),
            full((256, 128)), full((1, 128)), full((1, 128)), full((1, 128)),
            full((128, 1)), full((1, 1)),
        ],
        out_specs=pl.BlockSpec((_BT, 1), lambda i: (i, 0)),
        out_shape=jax.ShapeDtypeStruct((_BATCH, 1), jnp.float32),
    )(h, lin2d, W1, b1, g1, be1, W2, b2, g2, be2,
      W3, b3, g3, be3, Wo, bo)


def kernel(x, additional, linear_w, linear_b, emb,
           W1, b1, g1, be1, W2, b2, g2, be2, W3, b3, g3, be3, Wo, bo):
    del additional
    xi = (x[:, _KEPT_COLS].astype(jnp.int32)
          + jnp.asarray(_OFFSETS)[None, :])
    ev = xi[:, 0::2].T.reshape(_NW, _NCHUNK, _CHUNK // 2)
    od = xi[:, 1::2].T.reshape(_NW, _NCHUNK, _CHUNK // 2)
    idxq = jnp.stack([ev, od], axis=2)
    idxt = (xi.reshape(_NW, _SAMP_W, _NFIELD)
            .transpose(0, 2, 1).reshape(_NW, _IDX_W))

    rows, lin = _sc_gather_call(idxq, idxt, emb, linear_w.reshape(_TABLE))
    lin2d = lin.reshape(_BATCH, 1) + linear_b[0]

    bf = jnp.bfloat16
    out = _mlp_call(rows, lin2d,
                    W1.astype(bf), b1.reshape(1, -1), g1.reshape(1, -1),
                    be1.reshape(1, -1),
                    W2.astype(bf), b2.reshape(1, -1), g2.reshape(1, -1),
                    be2.reshape(1, -1),
                    W3.astype(bf), b3.reshape(1, -1), g3.reshape(1, -1),
                    be3.reshape(1, -1),
                    Wo.astype(bf), bo.reshape(1, 1))
    return out.reshape(_BATCH)
